# baseline (device time: 37017 ns/iter reference)
import jax
import jax.numpy as jnp
from jax import lax
from jax.experimental import pallas as pl
from jax.experimental.pallas import tpu as pltpu

N_DEV = 32
CAP = 6


def kernel(x, router_W, route_idx, expert_W):
    n_tok, d = x.shape
    n_loc, _, h = expert_W.shape
    E = N_DEV * n_loc
    S = n_loc * CAP
    hl = h // 8

    e = route_idx[:, 0].astype(jnp.int32)
    onehot = (e[:, None] == jnp.arange(E, dtype=jnp.int32)[None, :]).astype(
        jnp.int32
    )
    rank = jnp.sum(jnp.cumsum(onehot, axis=0) * onehot, axis=1) - 1
    keep = rank < CAP
    slot = jnp.where(keep, e * CAP + rank, -1)
    slot_oh = (
        slot[None, :] == jnp.arange(E * CAP, dtype=jnp.int32)[:, None]
    )
    dest = (
        jnp.sum(
            jnp.where(slot_oh, jnp.arange(1, n_tok + 1, dtype=jnp.int32), 0),
            axis=1,
        )
        - 1
    )
    dest = jnp.where(dest < 0, n_tok, dest)
    dest_tok = dest.reshape(N_DEV, S)

    def body(x_ref, ew_ref, dest_ref, out_ref, comm_ref, cx_ref, stage_ref,
             acc_ref, xv_ref, ewv_ref, copy_sems, send_sems, recv_sems):
        my_pos = lax.axis_index("i")

        cpx = pltpu.make_async_copy(x_ref, xv_ref, copy_sems.at[0])
        cpx.start()
        cpw = pltpu.make_async_copy(ew_ref, ewv_ref, copy_sems.at[1])
        cpw.start()

        barrier_sem = pltpu.get_barrier_semaphore()
        for off in range(1, N_DEV):
            pl.semaphore_signal(
                barrier_sem,
                inc=1,
                device_id=(lax.rem(my_pos + off, N_DEV),),
                device_id_type=pl.DeviceIdType.MESH,
            )

        cpx.wait()
        for j in range(S):
            tok = dest_ref[my_pos, j]

            @pl.when(tok < n_tok)
            def _():
                cx_ref[pl.ds(j, 1), :] = xv_ref[pl.ds(tok, 1), :]

        cpw.wait()
        for k in range(n_loc):
            r = lax.dot_general(
                cx_ref[CAP * k : CAP * (k + 1), :],
                ewv_ref[k],
                (((1,), (0,)), ((), ())),
                preferred_element_type=jnp.float32,
            )
            comm_ref[pl.ds(my_pos, 1), pl.ds(CAP * k, CAP), :] = r.astype(
                jnp.bfloat16
            )[None]

        pl.semaphore_wait(barrier_sem, N_DEV - 1)

        def send_to(tgt):
            return pltpu.make_async_remote_copy(
                src_ref=comm_ref.at[pl.ds(my_pos, 1)],
                dst_ref=comm_ref.at[pl.ds(my_pos, 1)],
                send_sem=send_sems.at[tgt],
                recv_sem=recv_sems.at[my_pos],
                device_id=(tgt,),
                device_id_type=pl.DeviceIdType.MESH,
            )

        def recv_from(origin):
            return pltpu.make_async_remote_copy(
                src_ref=comm_ref.at[pl.ds(origin, 1)],
                dst_ref=comm_ref.at[pl.ds(origin, 1)],
                send_sem=send_sems.at[origin],
                recv_sem=recv_sems.at[origin],
                device_id=(origin,),
                device_id_type=pl.DeviceIdType.MESH,
            )

        for off in range(1, N_DEV):
            send_to(lax.rem(my_pos + off, N_DEV)).start()

        acc_ref[:, :] = jnp.zeros((n_tok + 8, h), jnp.float32)

        def scatter_chunk(s):
            stage_ref[:, :] = comm_ref[pl.ds(s, 1)][0].astype(jnp.float32)
            for j in range(S):
                dtok = dest_ref[s, j]
                acc_ref[pl.ds(dtok, 1), :] = stage_ref[pl.ds(j, 1), :]

        scatter_chunk(my_pos)

        for off in range(1, N_DEV):
            origin = lax.rem(my_pos - off + N_DEV, N_DEV)
            recv_from(origin).wait_recv()
            scatter_chunk(origin)

        cpo = pltpu.make_async_copy(
            acc_ref.at[pl.ds(0, n_tok)], out_ref, copy_sems.at[2]
        )
        cpo.start()
        cpo.wait()

        for off in range(1, N_DEV):
            send_to(lax.rem(my_pos + off, N_DEV)).wait_send()

    return pl.pallas_call(
        body,
        out_shape=jax.ShapeDtypeStruct((n_tok, h), jnp.float32),
        in_specs=[
            pl.BlockSpec(memory_space=pl.ANY),
            pl.BlockSpec(memory_space=pl.ANY),
            pl.BlockSpec(memory_space=pltpu.SMEM),
        ],
        out_specs=pl.BlockSpec(memory_space=pl.ANY),
        scratch_shapes=[
            pltpu.VMEM((N_DEV, S, h), jnp.bfloat16),
            pltpu.VMEM((S, d), jnp.float32),
            pltpu.VMEM((S, h), jnp.float32),
            pltpu.VMEM((n_tok + 8, h), jnp.float32),
            pltpu.VMEM((n_tok, d), jnp.float32),
            pltpu.VMEM((n_loc, d, h), jnp.float32),
            pltpu.SemaphoreType.DMA((3,)),
            pltpu.SemaphoreType.DMA((N_DEV,)),
            pltpu.SemaphoreType.DMA((N_DEV,)),
        ],
        compiler_params=pltpu.CompilerParams(collective_id=0),
    )(x, expert_W, dest_tok)


# device time: 36228 ns/iter; 1.0218x vs baseline; 1.0218x over previous
import jax
import jax.numpy as jnp
from jax import lax
from jax.experimental import pallas as pl
from jax.experimental.pallas import tpu as pltpu

N_DEV = 32
CAP = 6


def kernel(x, router_W, route_idx, expert_W):
    n_tok, d = x.shape
    n_loc, _, h = expert_W.shape
    E = N_DEV * n_loc
    S = n_loc * CAP
    hl = h // 8

    e = route_idx[:, 0].astype(jnp.int32)
    onehot = (e[:, None] == jnp.arange(E, dtype=jnp.int32)[None, :]).astype(
        jnp.int32
    )
    rank = jnp.sum(jnp.cumsum(onehot, axis=0) * onehot, axis=1) - 1
    keep = rank < CAP
    slot = jnp.where(keep, e * CAP + rank, -1)
    slot_oh = (
        slot[None, :] == jnp.arange(E * CAP, dtype=jnp.int32)[:, None]
    )
    dest = (
        jnp.sum(
            jnp.where(slot_oh, jnp.arange(1, n_tok + 1, dtype=jnp.int32), 0),
            axis=1,
        )
        - 1
    )
    dest = jnp.where(dest < 0, n_tok, dest)
    dest_tok = dest.reshape(N_DEV, S)

    def body(x_ref, ew_ref, dest_ref, out_ref, comm_ref, cx_ref, stage_ref,
             acc_ref, send_sems, recv_sems):
        my_pos = lax.axis_index("i")

        barrier_sem = pltpu.get_barrier_semaphore()
        for off in range(1, N_DEV):
            pl.semaphore_signal(
                barrier_sem,
                inc=1,
                device_id=(lax.rem(my_pos + off, N_DEV),),
                device_id_type=pl.DeviceIdType.MESH,
            )

        for j in range(S):
            tok = dest_ref[my_pos, j]

            @pl.when(tok < n_tok)
            def _():
                cx_ref[pl.ds(j, 1), :] = x_ref[pl.ds(tok, 1), :]

        for k in range(n_loc):
            r = lax.dot_general(
                cx_ref[CAP * k : CAP * (k + 1), :],
                ew_ref[k],
                (((1,), (0,)), ((), ())),
                preferred_element_type=jnp.float32,
            )
            comm_ref[pl.ds(my_pos, 1), pl.ds(CAP * k, CAP), :] = r.astype(
                jnp.bfloat16
            )[None]

        pl.semaphore_wait(barrier_sem, N_DEV - 1)

        def send_to(tgt):
            return pltpu.make_async_remote_copy(
                src_ref=comm_ref.at[pl.ds(my_pos, 1)],
                dst_ref=comm_ref.at[pl.ds(my_pos, 1)],
                send_sem=send_sems.at[tgt],
                recv_sem=recv_sems.at[my_pos],
                device_id=(tgt,),
                device_id_type=pl.DeviceIdType.MESH,
            )

        def recv_from(origin):
            return pltpu.make_async_remote_copy(
                src_ref=comm_ref.at[pl.ds(origin, 1)],
                dst_ref=comm_ref.at[pl.ds(origin, 1)],
                send_sem=send_sems.at[origin],
                recv_sem=recv_sems.at[origin],
                device_id=(origin,),
                device_id_type=pl.DeviceIdType.MESH,
            )

        for off in range(1, N_DEV):
            send_to(lax.rem(my_pos + off, N_DEV)).start()

        acc_ref[:, :] = jnp.zeros((n_tok + 8, h), jnp.float32)

        def scatter_chunk(s):
            stage_ref[:, :] = comm_ref[pl.ds(s, 1)][0].astype(jnp.float32)
            for j in range(S):
                dtok = dest_ref[s, j]
                acc_ref[pl.ds(dtok, 1), :] = stage_ref[pl.ds(j, 1), :]

        scatter_chunk(my_pos)

        for off in range(1, N_DEV):
            origin = lax.rem(my_pos - off + N_DEV, N_DEV)
            recv_from(origin).wait_recv()
            scatter_chunk(origin)

        out_ref[:, :] = acc_ref[pl.ds(0, n_tok), :]

        for off in range(1, N_DEV):
            send_to(lax.rem(my_pos + off, N_DEV)).wait_send()

    return pl.pallas_call(
        body,
        out_shape=jax.ShapeDtypeStruct((n_tok, h), jnp.float32),
        in_specs=[
            pl.BlockSpec(memory_space=pltpu.VMEM),
            pl.BlockSpec(memory_space=pltpu.VMEM),
            pl.BlockSpec(memory_space=pltpu.SMEM),
        ],
        out_specs=pl.BlockSpec(memory_space=pltpu.VMEM),
        scratch_shapes=[
            pltpu.VMEM((N_DEV, S, h), jnp.bfloat16),
            pltpu.VMEM((S, d), jnp.float32),
            pltpu.VMEM((S, h), jnp.float32),
            pltpu.VMEM((n_tok + 8, h), jnp.float32),
            pltpu.SemaphoreType.DMA((N_DEV,)),
            pltpu.SemaphoreType.DMA((N_DEV,)),
        ],
        compiler_params=pltpu.CompilerParams(collective_id=0),
    )(x, expert_W, dest_tok)
